# Initial kernel scaffold; baseline (speedup 1.0000x reference)
import jax
import jax.numpy as jnp
from jax.experimental import pallas as pl


def kernel(h, pos, dis, syn, edge_index, W_pos, b_pos, W_dis, b_dis, W_syn, b_syn):
    raise NotImplementedError("write your pallas kernel here")


# speedup vs baseline: 1.5058x; 1.5058x over previous
import jax
import jax.numpy as jnp
from jax.experimental import pallas as pl
from jax.experimental.pallas import tpu as pltpu


def _fused_out_kernel(agg_ref, f_ref, h_ref, W_ref, b_ref, o_ref):
    x = f_ref[...]
    bias = jax.lax.dot_general(
        x, W_ref[...], (((1,), (1,)), ((), ())),
        precision=jax.lax.Precision.HIGHEST,
        preferred_element_type=jnp.float32)
    o_ref[...] = agg_ref[...] + bias + b_ref[...] + h_ref[...]


def _fused_out(agg, f, h, W, b):
    n, dm = f.shape
    blk = 1000
    return pl.pallas_call(
        _fused_out_kernel,
        grid=(n // blk,),
        in_specs=[
            pl.BlockSpec((blk, dm), lambda i: (i, 0)),
            pl.BlockSpec((blk, dm), lambda i: (i, 0)),
            pl.BlockSpec((blk, dm), lambda i: (i, 0)),
            pl.BlockSpec((dm, dm), lambda i: (0, 0)),
            pl.BlockSpec((1, dm), lambda i: (0, 0)),
        ],
        out_specs=pl.BlockSpec((blk, dm), lambda i: (i, 0)),
        out_shape=jax.ShapeDtypeStruct((n, dm), jnp.float32),
    )(agg, f, h, W, b)


def kernel(h, pos, dis, syn, edge_index, W_pos, b_pos, W_dis, b_dis, W_syn, b_syn):
    n = h.shape[0]
    src = edge_index[0]
    dst = edge_index[1]

    def dots(f):
        return jnp.sum(f[src] * f[dst], axis=-1)

    s_pos, s_dis, s_syn = dots(pos), dots(dis), dots(syn)

    def soft(s):
        e = jnp.exp(jnp.minimum(s, 80.0))
        den = jax.ops.segment_sum(e, dst, num_segments=n)
        return e / den[dst]

    a_pos, a_dis, a_syn = soft(s_pos), soft(s_dis), soft(s_syn)
    ep, ed, es = jnp.exp(s_pos), jnp.exp(s_dis), jnp.exp(s_syn)
    dc = ep + ed + es
    w_pos = a_pos + ep / dc
    w_dis = a_dis + ed / dc
    w_syn = a_syn + es / dc

    hs = h[src]

    def agg(w):
        return jax.ops.segment_sum(hs * w[:, None], dst, num_segments=n)

    outs = []
    for f, W, b, w in ((pos, W_pos, b_pos, w_pos),
                       (dis, W_dis, b_dis, w_dis),
                       (syn, W_syn, b_syn, w_syn)):
        outs.append(_fused_out(agg(w), f, h, W, b.reshape(1, -1)))
    return tuple(outs)


# speedup vs baseline: 4.1062x; 2.7269x over previous
import dataclasses
import functools

import jax
import jax.numpy as jnp
from jax import lax
from jax.experimental import pallas as pl
from jax.experimental.pallas import tpu as pltpu
from jax.experimental.pallas import tpu_sc as plsc

NC, NS, L = 2, 16, 16
NW = NC * NS
N = 10000
D = 256
FD = 3 * D
E_PAD = 163840
EPW = E_PAD // NW
NDEN = 10240
SLN = NDEN // NW
B1 = 16
SC_STAGE = 1024
K2B = 512
CAPM = 2048
GB = 32
ACC_R = 128


@functools.cache
def _get_mesh():
    return plsc.VectorSubcoreMesh(core_axis_name="c", subcore_axis_name="s",
                                  num_cores=NC, num_subcores=NS)


_cp = pltpu.CompilerParams()
if "needs_layout_passes" in pltpu.CompilerParams.__dataclass_fields__:
    _cp = dataclasses.replace(_cp, needs_layout_passes=False)


def _wid():
    return lax.axis_index("s") * NC + lax.axis_index("c")


def _k1_body(f_hbm, src_hbm, dst_hbm, sc6_hbm, denp_hbm,
             srcall, dstall, fs0, fd0, fs1, fd1, ixs0, ixd0, ixs1, ixd1,
             part0, part1, part2, st0, st1, st2, st3, st4, st5,
             den0, den1, den2, sem0, sem1):
    parts = (part0, part1, part2)
    sts = (st0, st1, st2, st3, st4, st5)
    dens = (den0, den1, den2)
    w = _wid()
    base = w * EPW
    iota = lax.iota(jnp.int32, L)
    zf = jnp.zeros((L,), jnp.float32)

    @pl.loop(0, NDEN, step=L)
    def _(i):
        for ch in range(3):
            dens[ch][pl.ds(i, L)] = zf

    pltpu.sync_copy(src_hbm.at[pl.ds(base, EPW)], srcall)
    pltpu.sync_copy(dst_hbm.at[pl.ds(base, EPW)], dstall)

    def issue(bi, sel):
        fs, fd, ixs, ixd, sema = ((fs0, fd0, ixs0, ixd0, sem0) if sel == 0
                                  else (fs1, fd1, ixs1, ixd1, sem1))
        @pl.loop(0, B1, step=L)
        def _(q):
            ixs[pl.ds(q, L)] = srcall[pl.ds(bi * B1 + q, L)]
            ixd[pl.ds(q, L)] = dstall[pl.ds(bi * B1 + q, L)]
        pltpu.async_copy(f_hbm.at[ixs], fs, sema)
        pltpu.async_copy(f_hbm.at[ixd], fd, sema)

    def wait(sel):
        fs, fd, ixs, ixd, sema = ((fs0, fd0, ixs0, ixd0, sem0) if sel == 0
                                  else (fs1, fd1, ixs1, ixd1, sem1))
        pltpu.make_async_copy(f_hbm.at[ixs], fs, sema).wait()
        pltpu.make_async_copy(f_hbm.at[ixd], fd, sema).wait()

    def compute(bi, sel):
        fs, fd = (fs0, fd0) if sel == 0 else (fs1, fd1)
        for j in range(B1):
            def dotc(c, accs):
                a0, a1, a2 = accs
                a0 = a0 + fs[j, pl.ds(c * L, L)] * fd[j, pl.ds(c * L, L)]
                a1 = a1 + (fs[j, pl.ds(D + c * L, L)]
                           * fd[j, pl.ds(D + c * L, L)])
                a2 = a2 + (fs[j, pl.ds(2 * D + c * L, L)]
                           * fd[j, pl.ds(2 * D + c * L, L)])
                return (a0, a1, a2)
            a0, a1, a2 = lax.fori_loop(0, D // L, dotc, (zf, zf, zf))
            part0[j, :] = a0
            part1[j, :] = a1
            part2[j, :] = a2
        es, er = [], []
        for ch in range(3):
            dv = zf
            for c in range(L):
                dv = dv + plsc.load_gather(
                    parts[ch], [iota, jnp.full((L,), c, jnp.int32)])
            es.append(jnp.exp(jnp.minimum(dv, 80.0)))
            er.append(jnp.exp(dv))
        rdc = 1.0 / (er[0] + er[1] + er[2])
        soff = (bi % (SC_STAGE // B1)) * B1
        dstg = dstall[pl.ds(bi * B1, L)]
        for ch in range(3):
            sts[ch][pl.ds(soff, L)] = es[ch]
            sts[3 + ch][pl.ds(soff, L)] = er[ch] * rdc
            plsc.addupdate_scatter(dens[ch], [dstg], es[ch])

    nfl = EPW // SC_STAGE
    bpf = SC_STAGE // B1

    issue(0, 0)

    @pl.loop(0, nfl)
    def _(fi):
        @pl.loop(0, bpf // 2)
        def _(hi):
            bi = fi * bpf + hi * 2
            @pl.when(bi + 1 < EPW // B1)
            def _():
                issue(bi + 1, 1)
            wait(0)
            compute(bi, 0)
            @pl.when(bi + 2 < EPW // B1)
            def _():
                issue(bi + 2, 0)
            wait(1)
            compute(bi + 1, 1)
        for f in range(6):
            pltpu.sync_copy(
                sts[f],
                sc6_hbm.at[pl.ds(f * E_PAD + base + fi * SC_STAGE,
                                 SC_STAGE)])

    for ch in range(3):
        pltpu.sync_copy(dens[ch],
                        denp_hbm.at[pl.ds((w * 3 + ch) * NDEN, NDEN)])


def _k1(f, src_p, dst_p):
    kern = pl.kernel(
        _k1_body,
        out_type=[jax.ShapeDtypeStruct((6 * E_PAD,), jnp.float32),
                  jax.ShapeDtypeStruct((NW * 3 * NDEN,), jnp.float32)],
        mesh=_get_mesh(),
        compiler_params=_cp,
        scratch_types=([pltpu.VMEM((EPW,), jnp.int32),
                        pltpu.VMEM((EPW,), jnp.int32)]
                       + [pltpu.VMEM((B1, FD), jnp.float32)] * 4
                       + [pltpu.VMEM((B1,), jnp.int32)] * 4
                       + [pltpu.VMEM((L, L), jnp.float32)] * 3
                       + [pltpu.VMEM((SC_STAGE,), jnp.float32)] * 6
                       + [pltpu.VMEM((NDEN,), jnp.float32)] * 3
                       + [pltpu.SemaphoreType.DMA] * 2))
    return kern(f, src_p, dst_p)


def _k1r_body(denp_hbm, rden_hbm, acc0, acc1, acc2, tmp0, tmp1, tmp2):
    accs = (acc0, acc1, acc2)
    tmps = (tmp0, tmp1, tmp2)
    w = _wid()
    sl = w * SLN
    zf = jnp.zeros((L,), jnp.float32)

    @pl.loop(0, SLN, step=L)
    def _(i):
        for ch in range(3):
            accs[ch][pl.ds(i, L)] = zf

    @pl.loop(0, NW)
    def _(j):
        for ch in range(3):
            pltpu.sync_copy(
                denp_hbm.at[pl.ds((j * 3 + ch) * NDEN + sl, SLN)], tmps[ch])
        @pl.loop(0, SLN, step=L)
        def _(i):
            for ch in range(3):
                accs[ch][pl.ds(i, L)] = (accs[ch][pl.ds(i, L)]
                                         + tmps[ch][pl.ds(i, L)])

    @pl.loop(0, SLN, step=L)
    def _(i):
        for ch in range(3):
            accs[ch][pl.ds(i, L)] = 1.0 / accs[ch][pl.ds(i, L)]

    for ch in range(3):
        pltpu.sync_copy(accs[ch], rden_hbm.at[pl.ds(ch * NDEN + sl, SLN)])


def _k1r(denp):
    kern = pl.kernel(
        _k1r_body,
        out_type=jax.ShapeDtypeStruct((3 * NDEN,), jnp.float32),
        mesh=_get_mesh(),
        compiler_params=_cp,
        scratch_types=[pltpu.VMEM((SLN,), jnp.float32)] * 6)
    return kern(denp)


_K1WB = 1024


def _k1w_body(sc6_hbm, dst_hbm, rden_hbm, w3_hbm,
              rv0, rv1, rv2, sv0, sv1, sv2, sv3, sv4, sv5,
              dst_v, wv0, wv1, wv2):
    rvs = (rv0, rv1, rv2)
    svs = (sv0, sv1, sv2, sv3, sv4, sv5)
    wvs = (wv0, wv1, wv2)
    w = _wid()
    base = w * EPW
    for ch in range(3):
        pltpu.sync_copy(rden_hbm.at[pl.ds(ch * NDEN, NDEN)], rvs[ch])

    @pl.loop(0, EPW // _K1WB)
    def _(bi):
        off = base + bi * _K1WB
        pltpu.sync_copy(dst_hbm.at[pl.ds(off, _K1WB)], dst_v)
        for f in range(6):
            pltpu.sync_copy(sc6_hbm.at[pl.ds(f * E_PAD + off, _K1WB)],
                            svs[f])
        @pl.loop(0, _K1WB, step=L)
        def _(i):
            dd = dst_v[pl.ds(i, L)]
            for ch in range(3):
                r = plsc.load_gather(rvs[ch], [dd])
                wvs[ch][pl.ds(i, L)] = (svs[ch][pl.ds(i, L)] * r
                                        + svs[3 + ch][pl.ds(i, L)])
        for ch in range(3):
            pltpu.sync_copy(wvs[ch],
                            w3_hbm.at[pl.ds(ch * E_PAD + off, _K1WB)])


def _k1w(sc6, dst_p, rden):
    kern = pl.kernel(
        _k1w_body,
        out_type=jax.ShapeDtypeStruct((3 * E_PAD,), jnp.float32),
        mesh=_get_mesh(),
        compiler_params=_cp,
        scratch_types=([pltpu.VMEM((NDEN,), jnp.float32)] * 3
                       + [pltpu.VMEM((_K1WB,), jnp.float32)] * 6
                       + [pltpu.VMEM((_K1WB,), jnp.int32)]
                       + [pltpu.VMEM((_K1WB,), jnp.float32)] * 3))
    return kern(sc6, dst_p, rden)


def _k2_body(src_hbm, dst_hbm, w3_hbm, h_hbm, agg_hbm,
             sd0, ss0, sw00, sw01, sw02, sd1, ss1, sw10, sw11, sw12,
             msrc, mdst, mw1, mw2, mw3, ixg, rows,
             acc0, acc1, acc2, semg, sem0, sem1):
    accs = (acc0, acc1, acc2)
    c = lax.axis_index("c")
    s = lax.axis_index("s")
    zf = jnp.zeros((L,), jnp.float32)
    zi = jnp.zeros((L,), jnp.int32)
    ones = jnp.ones((L,), jnp.bool_)
    nb = E_PAD // K2B

    def bufs(sel):
        if sel == 0:
            return sd0, ss0, (sw00, sw01, sw02), sem0
        return sd1, ss1, (sw10, sw11, sw12), sem1

    def issue(i, sel):
        sd, ss, sws, sema = bufs(sel)
        boff = i * K2B
        pltpu.async_copy(dst_hbm.at[pl.ds(boff, K2B)], sd, sema)
        pltpu.async_copy(src_hbm.at[pl.ds(boff, K2B)], ss, sema)
        for ch in range(3):
            pltpu.async_copy(w3_hbm.at[pl.ds(ch * E_PAD + boff, K2B)],
                             sws[ch], sema)

    def wait(i, sel):
        sd, ss, sws, sema = bufs(sel)
        boff = i * K2B
        pltpu.make_async_copy(dst_hbm.at[pl.ds(boff, K2B)], sd, sema).wait()
        pltpu.make_async_copy(src_hbm.at[pl.ds(boff, K2B)], ss, sema).wait()
        for ch in range(3):
            pltpu.make_async_copy(
                w3_hbm.at[pl.ds(ch * E_PAD + boff, K2B)], sws[ch],
                sema).wait()

    for p in range(3):
        b = p * NC + c
        lo = b * (NS * ACC_R) + s * ACC_R

        @pl.loop(0, ACC_R)
        def _(i):
            @pl.loop(0, D, step=L)
            def _(jj):
                for ch in range(3):
                    accs[ch][i, pl.ds(jj, L)] = zf

        def flush(off):
            for q in range(GB // L):
                plsc.store_compressed(msrc.at[pl.ds(off + q * L, L)], zi,
                                      mask=ones)
                plsc.store_compressed(mdst.at[pl.ds(off + q * L, L)], zi,
                                      mask=ones)
                plsc.store_compressed(mw1.at[pl.ds(off + q * L, L)], zf,
                                      mask=ones)
                plsc.store_compressed(mw2.at[pl.ds(off + q * L, L)], zf,
                                      mask=ones)
                plsc.store_compressed(mw3.at[pl.ds(off + q * L, L)], zf,
                                      mask=ones)
            ng = (off + GB - 1) // GB

            def group(g, carry):
                @pl.loop(0, GB, step=L)
                def _(q):
                    ixg[pl.ds(q, L)] = msrc[pl.ds(g * GB + q, L)]
                pltpu.async_copy(h_hbm.at[ixg], rows, semg).wait()
                @pl.loop(0, GB, step=L)
                def _(sg):
                    w1 = mw1[pl.ds(g * GB + sg, L)]
                    w2 = mw2[pl.ds(g * GB + sg, L)]
                    w3 = mw3[pl.ds(g * GB + sg, L)]
                    dos = mdst[pl.ds(g * GB + sg, L)]
                    for j in range(L):
                        do = dos[j]
                        wv1 = jnp.full((L,), w1[j], jnp.float32)
                        wv2 = jnp.full((L,), w2[j], jnp.float32)
                        wv3 = jnp.full((L,), w3[j], jnp.float32)
                        @pl.loop(0, D, step=L)
                        def _(cc):
                            rv = rows[sg + j, pl.ds(cc, L)]
                            plsc.addupdate(acc0.at[do, pl.ds(cc, L)],
                                           rv * wv1)
                            plsc.addupdate(acc1.at[do, pl.ds(cc, L)],
                                           rv * wv2)
                            plsc.addupdate(acc2.at[do, pl.ds(cc, L)],
                                           rv * wv3)
                return carry

            lax.fori_loop(0, ng, group, 0)

        def scan_one(i, off, sel):
            sd, ss, sws, _ = bufs(sel)

            def chunk(q, off):
                dd = sd[pl.ds(q * L, L)]
                m = (dd >= lo) & (dd < lo + ACC_R)
                pc = jnp.max(plsc.all_reduce_population_count(m))

                @pl.when(pc > 0)
                def _():
                    plsc.store_compressed(msrc.at[pl.ds(off, L)],
                                          ss[pl.ds(q * L, L)], mask=m)
                    plsc.store_compressed(mdst.at[pl.ds(off, L)], dd - lo,
                                          mask=m)
                    plsc.store_compressed(mw1.at[pl.ds(off, L)],
                                          sws[0][pl.ds(q * L, L)], mask=m)
                    plsc.store_compressed(mw2.at[pl.ds(off, L)],
                                          sws[1][pl.ds(q * L, L)], mask=m)
                    plsc.store_compressed(mw3.at[pl.ds(off, L)],
                                          sws[2][pl.ds(q * L, L)], mask=m)
                return off + pc

            off = lax.fori_loop(0, K2B // L, chunk, off)

            @pl.when(off >= CAPM - K2B)
            def _():
                flush(off)
            return jnp.where(off >= CAPM - K2B, 0, off)

        issue(0, 0)

        def pair(hi, off):
            i0 = hi * 2

            @pl.when(i0 + 1 < nb)
            def _():
                issue(i0 + 1, 1)
            wait(i0, 0)
            off = scan_one(i0, off, 0)

            @pl.when(i0 + 2 < nb)
            def _():
                issue(i0 + 2, 0)
            wait(i0 + 1, 1)
            off = scan_one(i0 + 1, off, 1)
            return off

        off_end = lax.fori_loop(0, nb // 2, pair, 0)

        @pl.when(off_end > 0)
        def _():
            flush(off_end)

        if p < 2:
            for ch in range(3):
                pltpu.sync_copy(accs[ch],
                                agg_hbm.at[pl.ds(ch * NDEN + lo, ACC_R)])
        else:
            @pl.when(lo < NDEN)
            def _():
                for ch in range(3):
                    pltpu.sync_copy(
                        accs[ch],
                        agg_hbm.at[pl.ds(ch * NDEN + lo, ACC_R)])


def _k2(src_p, dst_p, w3, h):
    kern = pl.kernel(
        _k2_body,
        out_type=jax.ShapeDtypeStruct((3 * NDEN, D), jnp.float32),
        mesh=_get_mesh(),
        compiler_params=_cp,
        scratch_types=([pltpu.VMEM((K2B,), jnp.int32)] * 2
                       + [pltpu.VMEM((K2B,), jnp.float32)] * 3
                       + [pltpu.VMEM((K2B,), jnp.int32)] * 2
                       + [pltpu.VMEM((K2B,), jnp.float32)] * 3
                       + [pltpu.VMEM((CAPM + GB,), jnp.int32)] * 2
                       + [pltpu.VMEM((CAPM + GB,), jnp.float32)] * 3
                       + [pltpu.VMEM((GB,), jnp.int32)]
                       + [pltpu.VMEM((GB, D), jnp.float32)]
                       + [pltpu.VMEM((ACC_R, D), jnp.float32)] * 3
                       + [pltpu.SemaphoreType.DMA] * 3))
    return kern(src_p, dst_p, w3, h)


def _fused_out_kernel(agg_ref, f_ref, h_ref, w_ref, b_ref, o_ref):
    x = f_ref[...]
    bias = jax.lax.dot_general(
        x, w_ref[...], (((1,), (1,)), ((), ())),
        precision=jax.lax.Precision.HIGHEST,
        preferred_element_type=jnp.float32)
    o_ref[...] = agg_ref[...] + bias + b_ref[...] + h_ref[...]


def _fused_out(agg, f, h, w, b):
    n, dm = f.shape
    blk = 1000
    return pl.pallas_call(
        _fused_out_kernel,
        grid=(n // blk,),
        in_specs=[
            pl.BlockSpec((blk, dm), lambda i: (i, 0)),
            pl.BlockSpec((blk, dm), lambda i: (i, 0)),
            pl.BlockSpec((blk, dm), lambda i: (i, 0)),
            pl.BlockSpec((dm, dm), lambda i: (0, 0)),
            pl.BlockSpec((1, dm), lambda i: (0, 0)),
        ],
        out_specs=pl.BlockSpec((blk, dm), lambda i: (i, 0)),
        out_shape=jax.ShapeDtypeStruct((n, dm), jnp.float32),
    )(agg, f, h, w, b)


def kernel(h, pos, dis, syn, edge_index, W_pos, b_pos, W_dis, b_dis,
           W_syn, b_syn):
    e = edge_index.shape[1]
    pad = E_PAD - e
    src_p = jnp.concatenate([edge_index[0], jnp.zeros((pad,), jnp.int32)])
    dst_p = jnp.concatenate([edge_index[1], jnp.full((pad,), N, jnp.int32)])
    f = jnp.concatenate([pos, dis, syn], axis=1)

    sc6, denp = _k1(f, src_p, dst_p)
    rden = _k1r(denp)
    w3 = _k1w(sc6, dst_p, rden)
    agg = _k2(src_p, dst_p, w3, h)

    outs = []
    for ch, (feat, w, b) in enumerate(((pos, W_pos, b_pos),
                                       (dis, W_dis, b_dis),
                                       (syn, W_syn, b_syn))):
        outs.append(_fused_out(agg[ch * NDEN:ch * NDEN + N], feat, h, w,
                               b.reshape(1, -1)))
    return tuple(outs)


# speedup vs baseline: 4.6673x; 1.1367x over previous
import dataclasses
import functools

import jax
import jax.numpy as jnp
from jax import lax
from jax.experimental import pallas as pl
from jax.experimental.pallas import tpu as pltpu
from jax.experimental.pallas import tpu_sc as plsc

NC, NS, L = 2, 16, 16
NW = NC * NS
N = 10000
D = 256
FD = 3 * D
E_PAD = 163840
EPW = E_PAD // NW
NDEN = 10240
SLN = NDEN // NW
B1 = 16
SC_STAGE = 1024
K2B = 512
CAPM = 2048
GB = 16
ACC_R = 128


@functools.cache
def _get_mesh():
    return plsc.VectorSubcoreMesh(core_axis_name="c", subcore_axis_name="s",
                                  num_cores=NC, num_subcores=NS)


_cp = pltpu.CompilerParams()
if "needs_layout_passes" in pltpu.CompilerParams.__dataclass_fields__:
    _cp = dataclasses.replace(_cp, needs_layout_passes=False)


def _wid():
    return lax.axis_index("s") * NC + lax.axis_index("c")


def _k1_body(f_hbm, src_hbm, dst_hbm, sc6_hbm, denp_hbm,
             srcall, dstall, fs0, fd0, fs1, fd1, ixs0, ixd0, ixs1, ixd1,
             part0, part1, part2, st0, st1, st2, st3, st4, st5,
             den0, den1, den2, sem0, sem1):
    parts = (part0, part1, part2)
    sts = (st0, st1, st2, st3, st4, st5)
    dens = (den0, den1, den2)
    w = _wid()
    base = w * EPW
    iota = lax.iota(jnp.int32, L)
    zf = jnp.zeros((L,), jnp.float32)

    @pl.loop(0, NDEN, step=L)
    def _(i):
        for ch in range(3):
            dens[ch][pl.ds(i, L)] = zf

    pltpu.sync_copy(src_hbm.at[pl.ds(base, EPW)], srcall)
    pltpu.sync_copy(dst_hbm.at[pl.ds(base, EPW)], dstall)

    def issue(bi, sel):
        fs, fd, ixs, ixd, sema = ((fs0, fd0, ixs0, ixd0, sem0) if sel == 0
                                  else (fs1, fd1, ixs1, ixd1, sem1))
        @pl.loop(0, B1, step=L)
        def _(q):
            ixs[pl.ds(q, L)] = srcall[pl.ds(bi * B1 + q, L)]
            ixd[pl.ds(q, L)] = dstall[pl.ds(bi * B1 + q, L)]
        pltpu.async_copy(f_hbm.at[ixs], fs, sema)
        pltpu.async_copy(f_hbm.at[ixd], fd, sema)

    def wait(sel):
        fs, fd, ixs, ixd, sema = ((fs0, fd0, ixs0, ixd0, sem0) if sel == 0
                                  else (fs1, fd1, ixs1, ixd1, sem1))
        pltpu.make_async_copy(f_hbm.at[ixs], fs, sema).wait()
        pltpu.make_async_copy(f_hbm.at[ixd], fd, sema).wait()

    def compute(bi, sel):
        fs, fd = (fs0, fd0) if sel == 0 else (fs1, fd1)
        for j in range(B1):
            def dotc(c, accs):
                a0, a1, a2 = accs
                a0 = a0 + fs[j, pl.ds(c * L, L)] * fd[j, pl.ds(c * L, L)]
                a1 = a1 + (fs[j, pl.ds(D + c * L, L)]
                           * fd[j, pl.ds(D + c * L, L)])
                a2 = a2 + (fs[j, pl.ds(2 * D + c * L, L)]
                           * fd[j, pl.ds(2 * D + c * L, L)])
                return (a0, a1, a2)
            a0, a1, a2 = lax.fori_loop(0, D // L, dotc, (zf, zf, zf))
            part0[j, :] = a0
            part1[j, :] = a1
            part2[j, :] = a2
        es, er = [], []
        for ch in range(3):
            dv = zf
            for c in range(L):
                dv = dv + plsc.load_gather(
                    parts[ch], [iota, jnp.full((L,), c, jnp.int32)])
            es.append(jnp.exp(jnp.minimum(dv, 80.0)))
            er.append(jnp.exp(dv))
        rdc = 1.0 / (er[0] + er[1] + er[2])
        soff = (bi % (SC_STAGE // B1)) * B1
        dstg = dstall[pl.ds(bi * B1, L)]
        for ch in range(3):
            sts[ch][pl.ds(soff, L)] = es[ch]
            sts[3 + ch][pl.ds(soff, L)] = er[ch] * rdc
            plsc.addupdate_scatter(dens[ch], [dstg], es[ch])

    nfl = EPW // SC_STAGE
    bpf = SC_STAGE // B1

    issue(0, 0)

    @pl.loop(0, nfl)
    def _(fi):
        @pl.loop(0, bpf // 2)
        def _(hi):
            bi = fi * bpf + hi * 2
            @pl.when(bi + 1 < EPW // B1)
            def _():
                issue(bi + 1, 1)
            wait(0)
            compute(bi, 0)
            @pl.when(bi + 2 < EPW // B1)
            def _():
                issue(bi + 2, 0)
            wait(1)
            compute(bi + 1, 1)
        for f in range(6):
            pltpu.sync_copy(
                sts[f],
                sc6_hbm.at[pl.ds(f * E_PAD + base + fi * SC_STAGE,
                                 SC_STAGE)])

    for ch in range(3):
        pltpu.sync_copy(dens[ch],
                        denp_hbm.at[pl.ds((w * 3 + ch) * NDEN, NDEN)])


def _k1(f, src_p, dst_p):
    kern = pl.kernel(
        _k1_body,
        out_type=[jax.ShapeDtypeStruct((6 * E_PAD,), jnp.float32),
                  jax.ShapeDtypeStruct((NW * 3 * NDEN,), jnp.float32)],
        mesh=_get_mesh(),
        compiler_params=_cp,
        scratch_types=([pltpu.VMEM((EPW,), jnp.int32),
                        pltpu.VMEM((EPW,), jnp.int32)]
                       + [pltpu.VMEM((B1, FD), jnp.float32)] * 4
                       + [pltpu.VMEM((B1,), jnp.int32)] * 4
                       + [pltpu.VMEM((L, L), jnp.float32)] * 3
                       + [pltpu.VMEM((SC_STAGE,), jnp.float32)] * 6
                       + [pltpu.VMEM((NDEN,), jnp.float32)] * 3
                       + [pltpu.SemaphoreType.DMA] * 2))
    return kern(f, src_p, dst_p)


def _k1r_body(denp_hbm, rden_hbm, acc0, acc1, acc2, tmp0, tmp1, tmp2):
    accs = (acc0, acc1, acc2)
    tmps = (tmp0, tmp1, tmp2)
    w = _wid()
    sl = w * SLN
    zf = jnp.zeros((L,), jnp.float32)

    @pl.loop(0, SLN, step=L)
    def _(i):
        for ch in range(3):
            accs[ch][pl.ds(i, L)] = zf

    @pl.loop(0, NW)
    def _(j):
        for ch in range(3):
            pltpu.sync_copy(
                denp_hbm.at[pl.ds((j * 3 + ch) * NDEN + sl, SLN)], tmps[ch])
        @pl.loop(0, SLN, step=L)
        def _(i):
            for ch in range(3):
                accs[ch][pl.ds(i, L)] = (accs[ch][pl.ds(i, L)]
                                         + tmps[ch][pl.ds(i, L)])

    @pl.loop(0, SLN, step=L)
    def _(i):
        for ch in range(3):
            accs[ch][pl.ds(i, L)] = 1.0 / accs[ch][pl.ds(i, L)]

    for ch in range(3):
        pltpu.sync_copy(accs[ch], rden_hbm.at[pl.ds(ch * NDEN + sl, SLN)])


def _k1r(denp):
    kern = pl.kernel(
        _k1r_body,
        out_type=jax.ShapeDtypeStruct((3 * NDEN,), jnp.float32),
        mesh=_get_mesh(),
        compiler_params=_cp,
        scratch_types=[pltpu.VMEM((SLN,), jnp.float32)] * 6)
    return kern(denp)


_K1WB = 1024


def _k1w_body(sc6_hbm, dst_hbm, rden_hbm, w3_hbm,
              rv0, rv1, rv2, sv0, sv1, sv2, sv3, sv4, sv5,
              dst_v, wv0, wv1, wv2):
    rvs = (rv0, rv1, rv2)
    svs = (sv0, sv1, sv2, sv3, sv4, sv5)
    wvs = (wv0, wv1, wv2)
    w = _wid()
    base = w * EPW
    for ch in range(3):
        pltpu.sync_copy(rden_hbm.at[pl.ds(ch * NDEN, NDEN)], rvs[ch])

    @pl.loop(0, EPW // _K1WB)
    def _(bi):
        off = base + bi * _K1WB
        pltpu.sync_copy(dst_hbm.at[pl.ds(off, _K1WB)], dst_v)
        for f in range(6):
            pltpu.sync_copy(sc6_hbm.at[pl.ds(f * E_PAD + off, _K1WB)],
                            svs[f])
        @pl.loop(0, _K1WB, step=L)
        def _(i):
            dd = dst_v[pl.ds(i, L)]
            for ch in range(3):
                r = plsc.load_gather(rvs[ch], [dd])
                wvs[ch][pl.ds(i, L)] = (svs[ch][pl.ds(i, L)] * r
                                        + svs[3 + ch][pl.ds(i, L)])
        for ch in range(3):
            pltpu.sync_copy(wvs[ch],
                            w3_hbm.at[pl.ds(ch * E_PAD + off, _K1WB)])


def _k1w(sc6, dst_p, rden):
    kern = pl.kernel(
        _k1w_body,
        out_type=jax.ShapeDtypeStruct((3 * E_PAD,), jnp.float32),
        mesh=_get_mesh(),
        compiler_params=_cp,
        scratch_types=([pltpu.VMEM((NDEN,), jnp.float32)] * 3
                       + [pltpu.VMEM((_K1WB,), jnp.float32)] * 6
                       + [pltpu.VMEM((_K1WB,), jnp.int32)]
                       + [pltpu.VMEM((_K1WB,), jnp.float32)] * 3))
    return kern(sc6, dst_p, rden)


def _k2_body(src_hbm, dst_hbm, w3_hbm, h_hbm, agg_hbm,
             sd0, ss0, sw00, sw01, sw02, sd1, ss1, sw10, sw11, sw12,
             msrc, mdst, mw1, mw2, mw3, ixg0, ixg1, rows0, rows1,
             acc0, acc1, acc2, semg0, semg1, sem0, sem1):
    accs = (acc0, acc1, acc2)
    c = lax.axis_index("c")
    s = lax.axis_index("s")
    zf = jnp.zeros((L,), jnp.float32)
    zi = jnp.zeros((L,), jnp.int32)
    ones = jnp.ones((L,), jnp.bool_)
    nb = E_PAD // K2B

    def bufs(sel):
        if sel == 0:
            return sd0, ss0, (sw00, sw01, sw02), sem0
        return sd1, ss1, (sw10, sw11, sw12), sem1

    def issue(i, sel):
        sd, ss, sws, sema = bufs(sel)
        boff = i * K2B
        pltpu.async_copy(dst_hbm.at[pl.ds(boff, K2B)], sd, sema)
        pltpu.async_copy(src_hbm.at[pl.ds(boff, K2B)], ss, sema)
        for ch in range(3):
            pltpu.async_copy(w3_hbm.at[pl.ds(ch * E_PAD + boff, K2B)],
                             sws[ch], sema)

    def wait(i, sel):
        sd, ss, sws, sema = bufs(sel)
        boff = i * K2B
        pltpu.make_async_copy(dst_hbm.at[pl.ds(boff, K2B)], sd, sema).wait()
        pltpu.make_async_copy(src_hbm.at[pl.ds(boff, K2B)], ss, sema).wait()
        for ch in range(3):
            pltpu.make_async_copy(
                w3_hbm.at[pl.ds(ch * E_PAD + boff, K2B)], sws[ch],
                sema).wait()

    for p in range(3):
        b = p * NC + c
        lo = b * (NS * ACC_R) + s * ACC_R

        @pl.loop(0, ACC_R)
        def _(i):
            @pl.loop(0, D, step=L)
            def _(jj):
                for ch in range(3):
                    accs[ch][i, pl.ds(jj, L)] = zf

        def flush(off):
            plsc.store_compressed(msrc.at[pl.ds(off, L)], zi, mask=ones)
            plsc.store_compressed(mdst.at[pl.ds(off, L)], zi, mask=ones)
            plsc.store_compressed(mw1.at[pl.ds(off, L)], zf, mask=ones)
            plsc.store_compressed(mw2.at[pl.ds(off, L)], zf, mask=ones)
            plsc.store_compressed(mw3.at[pl.ds(off, L)], zf, mask=ones)
            ng = (off + GB - 1) // GB

            def gissue(g, sel):
                ixg, rows, semg = ((ixg0, rows0, semg0) if sel == 0
                                   else (ixg1, rows1, semg1))
                ixg[pl.ds(0, L)] = msrc[pl.ds(g * GB, L)]
                pltpu.async_copy(h_hbm.at[ixg], rows, semg)

            def gprocess(g, sel):
                ixg, rows, semg = ((ixg0, rows0, semg0) if sel == 0
                                   else (ixg1, rows1, semg1))
                pltpu.make_async_copy(h_hbm.at[ixg], rows, semg).wait()
                w1 = mw1[pl.ds(g * GB, L)]
                w2 = mw2[pl.ds(g * GB, L)]
                w3 = mw3[pl.ds(g * GB, L)]
                dos = mdst[pl.ds(g * GB, L)]
                for j in range(L):
                    do = dos[j]
                    wv1 = jnp.full((L,), w1[j], jnp.float32)
                    wv2 = jnp.full((L,), w2[j], jnp.float32)
                    wv3 = jnp.full((L,), w3[j], jnp.float32)
                    @pl.loop(0, D, step=L)
                    def _(cc):
                        rv = rows[j, pl.ds(cc, L)]
                        plsc.addupdate(acc0.at[do, pl.ds(cc, L)],
                                       rv * wv1)
                        plsc.addupdate(acc1.at[do, pl.ds(cc, L)],
                                       rv * wv2)
                        plsc.addupdate(acc2.at[do, pl.ds(cc, L)],
                                       rv * wv3)

            gissue(0, 0)

            def gpair(i, carry):
                g0 = 2 * i

                @pl.when(g0 + 1 < ng)
                def _():
                    gissue(g0 + 1, 1)
                gprocess(g0, 0)

                @pl.when(g0 + 2 < ng)
                def _():
                    gissue(g0 + 2, 0)

                @pl.when(g0 + 1 < ng)
                def _():
                    gprocess(g0 + 1, 1)
                return carry

            lax.fori_loop(0, (ng + 1) // 2, gpair, 0)

        def scan_one(i, off, sel):
            sd, ss, sws, _ = bufs(sel)

            def chunk(q, off):
                dd = sd[pl.ds(q * L, L)]
                m = (dd >= lo) & (dd < lo + ACC_R)
                pc = jnp.max(plsc.all_reduce_population_count(m))

                @pl.when(pc > 0)
                def _():
                    plsc.store_compressed(msrc.at[pl.ds(off, L)],
                                          ss[pl.ds(q * L, L)], mask=m)
                    plsc.store_compressed(mdst.at[pl.ds(off, L)], dd - lo,
                                          mask=m)
                    plsc.store_compressed(mw1.at[pl.ds(off, L)],
                                          sws[0][pl.ds(q * L, L)], mask=m)
                    plsc.store_compressed(mw2.at[pl.ds(off, L)],
                                          sws[1][pl.ds(q * L, L)], mask=m)
                    plsc.store_compressed(mw3.at[pl.ds(off, L)],
                                          sws[2][pl.ds(q * L, L)], mask=m)
                return off + pc

            off = lax.fori_loop(0, K2B // L, chunk, off)

            @pl.when(off >= CAPM - K2B)
            def _():
                flush(off)
            return jnp.where(off >= CAPM - K2B, 0, off)

        issue(0, 0)

        def pair(hi, off):
            i0 = hi * 2

            @pl.when(i0 + 1 < nb)
            def _():
                issue(i0 + 1, 1)
            wait(i0, 0)
            off = scan_one(i0, off, 0)

            @pl.when(i0 + 2 < nb)
            def _():
                issue(i0 + 2, 0)
            wait(i0 + 1, 1)
            off = scan_one(i0 + 1, off, 1)
            return off

        off_end = lax.fori_loop(0, nb // 2, pair, 0)

        @pl.when(off_end > 0)
        def _():
            flush(off_end)

        if p < 2:
            for ch in range(3):
                pltpu.sync_copy(accs[ch],
                                agg_hbm.at[pl.ds(ch * NDEN + lo, ACC_R)])
        else:
            @pl.when(lo < NDEN)
            def _():
                for ch in range(3):
                    pltpu.sync_copy(
                        accs[ch],
                        agg_hbm.at[pl.ds(ch * NDEN + lo, ACC_R)])


def _k2(src_p, dst_p, w3, h):
    kern = pl.kernel(
        _k2_body,
        out_type=jax.ShapeDtypeStruct((3 * NDEN, D), jnp.float32),
        mesh=_get_mesh(),
        compiler_params=_cp,
        scratch_types=([pltpu.VMEM((K2B,), jnp.int32)] * 2
                       + [pltpu.VMEM((K2B,), jnp.float32)] * 3
                       + [pltpu.VMEM((K2B,), jnp.int32)] * 2
                       + [pltpu.VMEM((K2B,), jnp.float32)] * 3
                       + [pltpu.VMEM((CAPM + GB,), jnp.int32)] * 2
                       + [pltpu.VMEM((CAPM + GB,), jnp.float32)] * 3
                       + [pltpu.VMEM((GB,), jnp.int32)] * 2
                       + [pltpu.VMEM((GB, D), jnp.float32)] * 2
                       + [pltpu.VMEM((ACC_R, D), jnp.float32)] * 3
                       + [pltpu.SemaphoreType.DMA] * 4))
    return kern(src_p, dst_p, w3, h)


def _fused_out_kernel(agg_ref, f_ref, h_ref, w_ref, b_ref, o_ref):
    x = f_ref[...]
    bias = jax.lax.dot_general(
        x, w_ref[...], (((1,), (1,)), ((), ())),
        precision=jax.lax.Precision.HIGHEST,
        preferred_element_type=jnp.float32)
    o_ref[...] = agg_ref[...] + bias + b_ref[...] + h_ref[...]


def _fused_out(agg, f, h, w, b):
    n, dm = f.shape
    blk = 1000
    return pl.pallas_call(
        _fused_out_kernel,
        grid=(n // blk,),
        in_specs=[
            pl.BlockSpec((blk, dm), lambda i: (i, 0)),
            pl.BlockSpec((blk, dm), lambda i: (i, 0)),
            pl.BlockSpec((blk, dm), lambda i: (i, 0)),
            pl.BlockSpec((dm, dm), lambda i: (0, 0)),
            pl.BlockSpec((1, dm), lambda i: (0, 0)),
        ],
        out_specs=pl.BlockSpec((blk, dm), lambda i: (i, 0)),
        out_shape=jax.ShapeDtypeStruct((n, dm), jnp.float32),
    )(agg, f, h, w, b)


def kernel(h, pos, dis, syn, edge_index, W_pos, b_pos, W_dis, b_dis,
           W_syn, b_syn):
    e = edge_index.shape[1]
    pad = E_PAD - e
    src_p = jnp.concatenate([edge_index[0], jnp.zeros((pad,), jnp.int32)])
    dst_p = jnp.concatenate([edge_index[1], jnp.full((pad,), N, jnp.int32)])
    f = jnp.concatenate([pos, dis, syn], axis=1)

    sc6, denp = _k1(f, src_p, dst_p)
    rden = _k1r(denp)
    w3 = _k1w(sc6, dst_p, rden)
    agg = _k2(src_p, dst_p, w3, h)

    outs = []
    for ch, (feat, w, b) in enumerate(((pos, W_pos, b_pos),
                                       (dis, W_dis, b_dis),
                                       (syn, W_syn, b_syn))):
        outs.append(_fused_out(agg[ch * NDEN:ch * NDEN + N], feat, h, w,
                               b.reshape(1, -1)))
    return tuple(outs)
